# baseline (speedup 1.0000x reference)
import jax
import jax.numpy as jnp
from jax.experimental import pallas as pl
from jax.experimental.pallas import tpu as pltpu
from jax.experimental.pallas import tpu_sc as plsc

_INTERPRET = False


def _c2_body(cbt_ref, out_ref):
    cbt = cbt_ref[0]
    out_ref[0] = jnp.sum(cbt * cbt, axis=0, keepdims=True)


def _argmin_body(zpt_ref, cbt_ref, c2_ref, out_ref):
    m = pl.program_id(0)
    zpt = zpt_ref[0, 0]
    cbt = cbt_ref[...]
    mt = zpt.shape[1]
    zp2 = jnp.transpose(jnp.sum(zpt * zpt, axis=0, keepdims=True))
    c2 = c2_ref[...]
    mm2 = jax.lax.dot_general(
        zpt + zpt, cbt, (((0,), (0,)), ((), ())),
        preferred_element_type=jnp.float32)
    dists = zp2 - mm2 + c2
    idx = jnp.argmin(dists, axis=1)
    out_ref[0, pl.ds(m * mt, mt)] = (cbt.shape[1] - 1) - idx.astype(jnp.int32)


def _compute_indices_one(z4i, cbt_rev, c2):
    b, _, dpc, hw = z4i.shape
    n = b * hw
    v = cbt_rev.shape[1]
    mt = min(512, hw)
    mpb = hw // mt
    return pl.pallas_call(
        _argmin_body,
        grid=(n // mt,),
        in_specs=[
            pl.BlockSpec((1, 1, dpc, mt), lambda m: (m // mpb, 0, 0, m % mpb)),
            pl.BlockSpec((dpc, v), lambda m: (0, 0)),
            pl.BlockSpec((1, v), lambda m: (0, 0)),
        ],
        out_specs=pl.BlockSpec((1, n), lambda m: (0, 0)),
        out_shape=jax.ShapeDtypeStruct((1, n), jnp.int32),
        interpret=_INTERPRET,
    )(z4i, cbt_rev, c2)


def _sc_gather_rows(cb_pad, idx):
    n_idx = idx.shape[1]
    val_dim = cb_pad.shape[1]
    gw = 128
    mesh = plsc.VectorSubcoreMesh(core_axis_name="c", subcore_axis_name="s")

    @pl.kernel(
        out_type=jax.ShapeDtypeStruct((n_idx, val_dim), cb_pad.dtype),
        mesh=mesh,
    )
    def gather_kernel(x_hbm, i_hbm, o_hbm):
        def body(i_vmem, o_vmem):
            pltpu.sync_copy(x_hbm.at[i_vmem.at[0]], o_vmem)

        pltpu.emit_pipeline(
            body,
            grid=(n_idx // gw,),
            in_specs=[pl.BlockSpec((1, gw), index_map=lambda i: (0, i))],
            out_specs=[pl.BlockSpec((gw, val_dim), index_map=lambda i: (i, 0))],
            core_axis_name=("c", "s"),
            dimension_semantics=(pltpu.PARALLEL,),
        )(i_hbm, o_hbm)

    return gather_kernel(cb_pad, idx)


def _pack_commit_body(pad_ref, z_ref, q_ref, acc_ref):
    @pl.when(pl.program_id(0) == 0)
    def _():
        acc_ref[...] = jnp.zeros((1, 1), jnp.float32)

    x = pad_ref[0]
    dpc = z_ref.shape[2]
    qt = jnp.transpose(x[:, :dpc])
    q_ref[0, 0] = qt
    dz = z_ref[0, 0] - qt
    acc_ref[...] += jnp.sum(dz * dz).reshape(1, 1)


def _pack_commit_one(rows_pad, z4i):
    b, _, dpc, hw = z4i.shape
    pad3 = rows_pad.reshape(b, hw, rows_pad.shape[1])
    q4, acc = pl.pallas_call(
        _pack_commit_body,
        grid=(b,),
        in_specs=[
            pl.BlockSpec((1, hw, rows_pad.shape[1]), lambda m: (m, 0, 0)),
            pl.BlockSpec((1, 1, dpc, hw), lambda m: (m, 0, 0, 0)),
        ],
        out_specs=[
            pl.BlockSpec((1, 1, dpc, hw), lambda m: (m, 0, 0, 0)),
            pl.BlockSpec((1, 1), lambda m: (0, 0)),
        ],
        out_shape=[
            jax.ShapeDtypeStruct((b, 1, dpc, hw), jnp.float32),
            jax.ShapeDtypeStruct((1, 1), jnp.float32),
        ],
        interpret=_INTERPRET,
    )(pad3, z4i)
    return q4, acc


def kernel(z, codebooks):
    b, d, h, w = z.shape
    n_cb, v, dpc = codebooks.shape
    hw = h * w
    z4 = z.reshape(b, n_cb, dpc, hw)
    cbt_rev = jnp.transpose(codebooks, (0, 2, 1))[:, :, ::-1]
    c2_all = pl.pallas_call(
        _c2_body,
        grid=(n_cb,),
        in_specs=[pl.BlockSpec((1, dpc, v), lambda i: (i, 0, 0))],
        out_specs=pl.BlockSpec((1, 1, v), lambda i: (i, 0, 0)),
        out_shape=jax.ShapeDtypeStruct((n_cb, 1, v), jnp.float32),
        interpret=_INTERPRET,
    )(cbt_rev)
    cb_pad = jnp.pad(codebooks, ((0, 0), (0, 0), (0, 128 - dpc)))

    idx_parts = []
    q_parts = []
    acc_parts = []
    for i in range(n_cb):
        z4i = jax.lax.slice_in_dim(z4, i, i + 1, axis=1)
        idx_i = _compute_indices_one(z4i, cbt_rev[i], c2_all[i])
        rows_i = _sc_gather_rows(cb_pad[i], idx_i)
        q_i, acc_i = _pack_commit_one(rows_i, z4i)
        idx_parts.append(idx_i)
        q_parts.append(q_i)
        acc_parts.append(acc_i)

    idx = jnp.concatenate(idx_parts, axis=0)
    indices = idx.reshape(n_cb, b, h, w).transpose(1, 0, 2, 3)
    q4 = jnp.concatenate(q_parts, axis=1)
    quantized = q4.reshape(b, d, h, w)
    commitment = sum(acc_parts)[0, 0] / jnp.float32(z.size)
    return quantized, indices, commitment


# speedup vs baseline: 1.4384x; 1.4384x over previous
import jax
import jax.numpy as jnp
from jax.experimental import pallas as pl
from jax.experimental.pallas import tpu as pltpu
from jax.experimental.pallas import tpu_sc as plsc

_INTERPRET = False


def _c2_body(cbt_ref, out_ref):
    cbt = cbt_ref[0]
    out_ref[0] = jnp.sum(cbt * cbt, axis=0, keepdims=True)


def _argmin_body(zpt_ref, cbt_ref, c2_ref, out_ref):
    m = pl.program_id(1)
    zpt = zpt_ref[0, 0]
    cbt = cbt_ref[0]
    mt = zpt.shape[1]
    zp2 = jnp.transpose(jnp.sum(zpt * zpt, axis=0, keepdims=True))
    c2 = c2_ref[0]
    mm2 = jax.lax.dot_general(
        zpt + zpt, cbt, (((0,), (0,)), ((), ())),
        preferred_element_type=jnp.float32)
    dists = zp2 - mm2 + c2
    idx = jnp.argmin(dists, axis=1)
    out_ref[0, 0, pl.ds(m * mt, mt)] = (cbt.shape[1] - 1) - idx.astype(jnp.int32)


def _compute_indices(z4, cbt_all):
    b, n_cb, dpc, hw = z4.shape
    n = b * hw
    v = cbt_all.shape[2]
    cbt_rev = cbt_all[:, :, ::-1]
    c2_all = pl.pallas_call(
        _c2_body,
        grid=(n_cb,),
        in_specs=[pl.BlockSpec((1, dpc, v), lambda i: (i, 0, 0))],
        out_specs=pl.BlockSpec((1, 1, v), lambda i: (i, 0, 0)),
        out_shape=jax.ShapeDtypeStruct((n_cb, 1, v), jnp.float32),
        interpret=_INTERPRET,
    )(cbt_rev)
    mt = min(512, hw)
    mpb = hw // mt
    out = pl.pallas_call(
        _argmin_body,
        grid=(n_cb, n // mt),
        in_specs=[
            pl.BlockSpec((1, 1, dpc, mt), lambda i, m: (m // mpb, i, 0, m % mpb)),
            pl.BlockSpec((1, dpc, v), lambda i, m: (i, 0, 0)),
            pl.BlockSpec((1, 1, v), lambda i, m: (i, 0, 0)),
        ],
        out_specs=pl.BlockSpec((1, 1, n), lambda i, m: (i, 0, 0)),
        out_shape=jax.ShapeDtypeStruct((n_cb, 1, n), jnp.int32),
        interpret=_INTERPRET,
    )(z4, cbt_rev, c2_all)
    return out.reshape(n_cb, n)


def _sc_gather(cb_flat, gidx):
    rows, dpc = cb_flat.shape
    cb_pad = jnp.pad(cb_flat, ((0, 0), (0, 128 - dpc)))
    return _sc_gather_rows(cb_pad, gidx)


def _sc_gather_rows(cb_flat, gidx):
    n_idx = gidx.shape[0]
    val_dim = cb_flat.shape[1]
    gw = 256
    mesh = plsc.VectorSubcoreMesh(core_axis_name="c", subcore_axis_name="s")
    idx2 = gidx.reshape(1, n_idx)

    @pl.kernel(
        out_type=jax.ShapeDtypeStruct((n_idx, val_dim), cb_flat.dtype),
        mesh=mesh,
    )
    def gather_kernel(x_hbm, i_hbm, o_hbm):
        def body(i_vmem, o_vmem):
            pltpu.sync_copy(x_hbm.at[i_vmem.at[0]], o_vmem)

        pltpu.emit_pipeline(
            body,
            grid=(n_idx // gw,),
            in_specs=[pl.BlockSpec((1, gw), index_map=lambda i: (0, i))],
            out_specs=[pl.BlockSpec((gw, val_dim), index_map=lambda i: (i, 0))],
            core_axis_name=("c", "s"),
            dimension_semantics=(pltpu.PARALLEL,),
        )(i_hbm, o_hbm)

    return gather_kernel(cb_flat, idx2)


def _pack_commit_body(pad_ref, z_ref, q_ref, acc_ref):
    @pl.when(pl.program_id(0) == 0)
    def _():
        acc_ref[...] = jnp.zeros((1, 1), jnp.float32)

    x = pad_ref[0]
    hw = z_ref.shape[2]
    n_cb = x.shape[0] // hw
    dpc = z_ref.shape[1] // n_cb
    x3 = x.reshape(hw, n_cb, x.shape[1])
    parts = [jnp.transpose(x3[:, i, :dpc]) for i in range(n_cb)]
    qt = jnp.concatenate(parts, axis=0)
    q_ref[0] = qt
    dz = z_ref[0] - qt
    acc_ref[...] += jnp.sum(dz * dz).reshape(1, 1)


def _pack_commit(out_pad, z3):
    b, d, hw = z3.shape
    n_cb = out_pad.shape[0] // (b * hw)
    pad3 = out_pad.reshape(b, n_cb * hw, out_pad.shape[1])
    q3, acc = pl.pallas_call(
        _pack_commit_body,
        grid=(b,),
        in_specs=[
            pl.BlockSpec((1, n_cb * hw, out_pad.shape[1]), lambda m: (m, 0, 0)),
            pl.BlockSpec((1, d, hw), lambda m: (m, 0, 0)),
        ],
        out_specs=[
            pl.BlockSpec((1, d, hw), lambda m: (m, 0, 0)),
            pl.BlockSpec((1, 1), lambda m: (0, 0)),
        ],
        out_shape=[
            jax.ShapeDtypeStruct((b, d, hw), jnp.float32),
            jax.ShapeDtypeStruct((1, 1), jnp.float32),
        ],
        interpret=_INTERPRET,
    )(pad3, z3)
    return q3, (acc / jnp.float32(z3.size))[0, 0]


def kernel(z, codebooks):
    b, d, h, w = z.shape
    n_cb, v, dpc = codebooks.shape
    n = b * h * w
    hw = h * w
    z4 = z.reshape(b, n_cb, dpc, hw)
    cbt = jnp.transpose(codebooks, (0, 2, 1))
    idx = _compute_indices(z4, cbt)
    indices = idx.reshape(n_cb, b, h, w).transpose(1, 0, 2, 3)
    gidx = (idx.T + (jnp.arange(n_cb, dtype=jnp.int32) * v)[None, :]).reshape(-1)
    out_pad = _sc_gather(codebooks.reshape(n_cb * v, dpc), gidx)
    q3, commitment = _pack_commit(out_pad, z.reshape(b, d, hw))
    quantized = q3.reshape(b, d, h, w)
    return quantized, indices, commitment
